# baseline (speedup 1.0000x reference)
import functools

import jax
import jax.numpy as jnp
from jax import lax
from jax.experimental import pallas as pl
from jax.experimental.pallas import tpu as pltpu
from jax.experimental.pallas import tpu_sc as plsc

HUMAN_EXHALATION_FLOW = 0.0052
TIME_STEP = 1.0

NC = 2
NS = 16
NW = NC * NS
LANES = 16

BR = 512
ROW_W = 1536
ROW_E = 128


def _conv_body(x_ref, xt_ref, w_ref, b_ref, out_ref):
    p = lax.dot_general(
        x_ref[...].astype(jnp.bfloat16), w_ref[...], (((1,), (0,)), ((), ())),
        preferred_element_type=jnp.float32)
    pt = lax.dot_general(
        xt_ref[...].astype(jnp.bfloat16), w_ref[...], (((1,), (0,)), ((), ())),
        preferred_element_type=jnp.float32)
    p2 = jnp.concatenate([p[1:, 128:130], pt[0:1, 128:130]], axis=0)
    v = jnp.concatenate([p[:, 0:126], p[:, 126:128] + p2], axis=1)
    out_ref[...] = v + b_ref[...]


def _conv_vals(m5, w12, b2, nrows):
    nb = nrows // BR
    out = pl.pallas_call(
        _conv_body,
        grid=(nb,),
        in_specs=[
            pl.BlockSpec((BR, ROW_W), lambda i: (i, 0)),
            pl.BlockSpec((16, ROW_W), lambda i: ((i + 1) * (BR // 16), 0)),
            pl.BlockSpec((ROW_W, 130), lambda i: (0, 0)),
            pl.BlockSpec((1, 1), lambda i: (0, 0)),
        ],
        out_specs=pl.BlockSpec((BR, ROW_E), lambda i: (i, 0)),
        out_shape=jax.ShapeDtypeStruct((nrows, ROW_E), jnp.float32),
    )(m5, m5, w12, b2)
    return out.reshape(nrows * ROW_E)


def _band_weights(wk):
    wf = wk.reshape(36)
    k = jnp.arange(ROW_W)[:, None]
    j = jnp.arange(ROW_E)[None, :]
    t = k - 12 * j
    band = jnp.where((t >= 0) & (t < 36), wf[jnp.clip(t, 0, 35)], 0.0)
    k1 = jnp.arange(ROW_W)
    c0 = jnp.where(k1 < 12, wf[jnp.clip(24 + k1, 0, 35)], 0.0)[:, None]
    c1 = jnp.where(k1 < 24, wf[jnp.clip(12 + k1, 0, 35)], 0.0)[:, None]
    return jnp.concatenate([band, c0, c1], axis=1)


def _sc_scatter(conc, size, src, dst, vals, N2):
    N = conc.shape[0]
    E = src.shape[0]
    ep = E // NW

    mesh = plsc.VectorSubcoreMesh(
        core_axis_name="c", subcore_axis_name="s",
        num_cores=NC, num_subcores=NS)

    UNROLL = 5
    n_chunks = ep // LANES
    assert n_chunks % UNROLL == 0
    nz = N2 // LANES
    assert nz % UNROLL == 0

    def body(conc_hbm, size_hbm, src_hbm, dst_hbm, vals_hbm, part_hbm,
             conc_v, size_v, acc_v, src_v, dst_v, vals_v, sems):
        wid = lax.axis_index("s") * NC + lax.axis_index("c")
        base = wid * ep
        cps = [
            pltpu.async_copy(conc_hbm, conc_v, sems.at[0]),
            pltpu.async_copy(size_hbm, size_v, sems.at[1]),
            pltpu.async_copy(src_hbm.at[pl.ds(base, ep)], src_v, sems.at[2]),
            pltpu.async_copy(dst_hbm.at[pl.ds(base, ep)], dst_v, sems.at[3]),
            pltpu.async_copy(vals_hbm.at[pl.ds(base, ep)], vals_v, sems.at[4]),
        ]

        @plsc.parallel_loop(0, nz, step=1, unroll=UNROLL)
        def zero_body(i):
            acc_v[pl.ds(i * LANES, LANES)] = jnp.zeros((LANES,), jnp.float32)

        for cp in cps:
            cp.wait()

        @plsc.parallel_loop(0, n_chunks, step=1, unroll=UNROLL)
        def edge_body(i):
            sl = pl.ds(i * LANES, LANES)
            s = src_v[sl]
            d = dst_v[sl]
            v = vals_v[sl]
            cs = plsc.load_gather(conc_v, [s])
            szs = plsc.load_gather(size_v, [s])
            szd = plsc.load_gather(size_v, [d])
            contrib = jnp.where(s != d, v * cs * TIME_STEP,
                                jnp.zeros((LANES,), jnp.float32))
            plsc.addupdate_scatter(acc_v, [s], -contrib / szs)
            plsc.addupdate_scatter(acc_v, [d], contrib / szd)

        pltpu.sync_copy(acc_v, part_hbm.at[wid])

    fn = pl.kernel(
        body,
        out_type=jax.ShapeDtypeStruct((NW, N2), jnp.float32),
        mesh=mesh,
        compiler_params=pltpu.CompilerParams(needs_layout_passes=False),
        scratch_types=[
            pltpu.VMEM((N,), jnp.float32),
            pltpu.VMEM((N,), jnp.float32),
            pltpu.VMEM((N2,), jnp.float32),
            pltpu.VMEM((ep,), jnp.int32),
            pltpu.VMEM((ep,), jnp.int32),
            pltpu.VMEM((ep,), jnp.float32),
            pltpu.SemaphoreType.DMA((5,)),
        ],
    )
    return fn(conc, size, src, dst, vals)


def _final(partials, conc_p, people_p, size_p, N, N2):
    def body(part_ref, conc_ref, people_ref, size_ref, out_ref):
        nn = jnp.sum(part_ref[...], axis=0, keepdims=True)
        pex = HUMAN_EXHALATION_FLOW * people_ref[...] / size_ref[...]
        idx = lax.broadcasted_iota(jnp.int32, (1, N2), 1)
        mask = jnp.where(idx == N - 1, 0.0, 1.0)
        out_ref[...] = conc_ref[...] + (nn + pex * TIME_STEP) * mask

    return pl.pallas_call(
        body,
        out_shape=jax.ShapeDtypeStruct((1, N2), jnp.float32),
    )(partials, conc_p, people_p, size_p)


def kernel(origin_data, flow, edge_index, conv_w, conv_b):
    N = origin_data.shape[0]
    E = edge_index.shape[1]
    N2 = ((N + 2559) // 2560) * 2560

    conc = origin_data[:, -1, 0]
    people = origin_data[:, -1, 1]
    size = origin_data[:, -1, 2]

    nrows = ((E // ROW_E) + BR - 1) // BR * BR
    nrows_m = nrows + 16
    flat = flow.reshape((E + 2) * 12)
    m5 = jnp.pad(flat, (0, nrows_m * ROW_W - flat.shape[0])).reshape(
        nrows_m, ROW_W)
    wk = jnp.transpose(conv_w[0], (1, 2, 0)).reshape(3, 12)
    w12 = _band_weights(wk).astype(jnp.bfloat16)
    b2 = conv_b.reshape(1, 1)
    vals_full = _conv_vals(m5, w12, b2, nrows)

    src = edge_index[0]
    dst = edge_index[1]
    partials = _sc_scatter(conc, size, src, dst, vals_full, N2)

    pad = N2 - N
    conc_p = jnp.pad(conc, (0, pad)).reshape(1, N2)
    people_p = jnp.pad(people, (0, pad)).reshape(1, N2)
    size_p = jnp.pad(size, (0, pad), constant_values=1.0).reshape(1, N2)
    res_p = _final(partials, conc_p, people_p, size_p, N, N2)

    result = res_p[0, :N][:, None]
    edge_feat = jnp.concatenate(
        [edge_index.T.astype(jnp.float32), vals_full[:E, None]], axis=1)
    return (result, edge_feat)


# speedup vs baseline: 25.9883x; 25.9883x over previous
import functools

import jax
import jax.numpy as jnp
from jax import lax
from jax.experimental import pallas as pl
from jax.experimental.pallas import tpu as pltpu
from jax.experimental.pallas import tpu_sc as plsc

HUMAN_EXHALATION_FLOW = 0.0052
TIME_STEP = 1.0

NC = 2
NS = 16
NW = NC * NS
LANES = 16

B = 25600


def _conv_body(fa_ref, ft_ref, w_ref, b_ref, out_ref):
    pm = lax.dot_general(
        w_ref[...], fa_ref[...], (((1,), (0,)), ((), ())),
        preferred_element_type=jnp.float32)
    pt = lax.dot_general(
        w_ref[...], ft_ref[...], (((1,), (0,)), ((), ())),
        preferred_element_type=jnp.float32)
    p = jnp.concatenate([pm, pt], axis=1)
    v = p[0:1, 0:B] + p[1:2, 1:B + 1] + p[2:3, 2:B + 2]
    out_ref[...] = v + b_ref[...]


def _conv_vals(flowT, wk, b2, E):
    nb = E // B
    out = pl.pallas_call(
        _conv_body,
        grid=(nb,),
        in_specs=[
            pl.BlockSpec((12, B), lambda i: (0, i)),
            pl.BlockSpec((12, 128), lambda i: (0, (i + 1) * (B // 128))),
            pl.BlockSpec((3, 12), lambda i: (0, 0)),
            pl.BlockSpec((1, 1), lambda i: (0, 0)),
        ],
        out_specs=pl.BlockSpec((1, B), lambda i: (0, i)),
        out_shape=jax.ShapeDtypeStruct((1, E), jnp.float32),
    )(flowT, flowT, wk, b2)
    return out.reshape(E)


def _sc_scatter(conc, size, src, dst, vals, N2):
    N = conc.shape[0]
    E = src.shape[0]
    ep = E // NW

    mesh = plsc.VectorSubcoreMesh(
        core_axis_name="c", subcore_axis_name="s",
        num_cores=NC, num_subcores=NS)

    UNROLL = 5
    n_chunks = ep // LANES
    assert n_chunks % UNROLL == 0
    nz = N2 // LANES
    assert nz % UNROLL == 0

    def body(conc_hbm, size_hbm, src_hbm, dst_hbm, vals_hbm, part_hbm,
             conc_v, size_v, acc_v, src_v, dst_v, vals_v, sems):
        wid = lax.axis_index("s") * NC + lax.axis_index("c")
        base = wid * ep
        cps = [
            pltpu.async_copy(conc_hbm, conc_v, sems.at[0]),
            pltpu.async_copy(size_hbm, size_v, sems.at[1]),
            pltpu.async_copy(src_hbm.at[pl.ds(base, ep)], src_v, sems.at[2]),
            pltpu.async_copy(dst_hbm.at[pl.ds(base, ep)], dst_v, sems.at[3]),
            pltpu.async_copy(vals_hbm.at[pl.ds(base, ep)], vals_v, sems.at[4]),
        ]

        @plsc.parallel_loop(0, nz, step=1, unroll=UNROLL)
        def zero_body(i):
            acc_v[pl.ds(i * LANES, LANES)] = jnp.zeros((LANES,), jnp.float32)

        for cp in cps:
            cp.wait()

        @plsc.parallel_loop(0, n_chunks, step=1, unroll=UNROLL)
        def edge_body(i):
            sl = pl.ds(i * LANES, LANES)
            s = src_v[sl]
            d = dst_v[sl]
            v = vals_v[sl]
            cs = plsc.load_gather(conc_v, [s])
            szs = plsc.load_gather(size_v, [s])
            szd = plsc.load_gather(size_v, [d])
            contrib = jnp.where(s != d, v * cs * TIME_STEP,
                                jnp.zeros((LANES,), jnp.float32))
            plsc.addupdate_scatter(acc_v, [s], -contrib / szs)
            plsc.addupdate_scatter(acc_v, [d], contrib / szd)

        pltpu.sync_copy(acc_v, part_hbm.at[wid])

    fn = pl.kernel(
        body,
        out_type=jax.ShapeDtypeStruct((NW, N2), jnp.float32),
        mesh=mesh,
        compiler_params=pltpu.CompilerParams(needs_layout_passes=False),
        scratch_types=[
            pltpu.VMEM((N,), jnp.float32),
            pltpu.VMEM((N,), jnp.float32),
            pltpu.VMEM((N2,), jnp.float32),
            pltpu.VMEM((ep,), jnp.int32),
            pltpu.VMEM((ep,), jnp.int32),
            pltpu.VMEM((ep,), jnp.float32),
            pltpu.SemaphoreType.DMA((5,)),
        ],
    )
    return fn(conc, size, src, dst, vals)


def _final(partials, conc_p, people_p, size_p, N, N2):
    def body(part_ref, conc_ref, people_ref, size_ref, out_ref):
        nn = jnp.sum(part_ref[...], axis=0, keepdims=True)
        pex = HUMAN_EXHALATION_FLOW * people_ref[...] / size_ref[...]
        idx = lax.broadcasted_iota(jnp.int32, (1, N2), 1)
        mask = jnp.where(idx == N - 1, 0.0, 1.0)
        out_ref[...] = conc_ref[...] + (nn + pex * TIME_STEP) * mask

    return pl.pallas_call(
        body,
        out_shape=jax.ShapeDtypeStruct((1, N2), jnp.float32),
    )(partials, conc_p, people_p, size_p)


def kernel(origin_data, flow, edge_index, conv_w, conv_b):
    N = origin_data.shape[0]
    E = edge_index.shape[1]
    N2 = ((N + 2559) // 2560) * 2560

    conc = origin_data[:, -1, 0]
    people = origin_data[:, -1, 1]
    size = origin_data[:, -1, 2]

    flowT = jnp.transpose(flow.reshape(E + 2, 12)).astype(jnp.bfloat16)
    wk = jnp.transpose(conv_w[0], (1, 2, 0)).reshape(3, 12).astype(jnp.bfloat16)
    b2 = conv_b.reshape(1, 1)
    vals = _conv_vals(flowT, wk, b2, E)

    src = edge_index[0]
    dst = edge_index[1]
    partials = _sc_scatter(conc, size, src, dst, vals, N2)

    pad = N2 - N
    conc_p = jnp.pad(conc, (0, pad)).reshape(1, N2)
    people_p = jnp.pad(people, (0, pad)).reshape(1, N2)
    size_p = jnp.pad(size, (0, pad), constant_values=1.0).reshape(1, N2)
    res_p = _final(partials, conc_p, people_p, size_p, N, N2)

    result = res_p[0, :N][:, None]
    edge_feat = jnp.concatenate(
        [edge_index.T.astype(jnp.float32), vals[:, None]], axis=1)
    return (result, edge_feat)


# speedup vs baseline: 27.6129x; 1.0625x over previous
import functools

import jax
import jax.numpy as jnp
from jax import lax
from jax.experimental import pallas as pl
from jax.experimental.pallas import tpu as pltpu
from jax.experimental.pallas import tpu_sc as plsc

HUMAN_EXHALATION_FLOW = 0.0052
TIME_STEP = 1.0

NC = 2
NS = 16
NW = NC * NS
LANES = 16

B = 64000


def _conv_body(fa_ref, ft_ref, w_ref, b_ref, out_ref):
    pm = lax.dot_general(
        w_ref[...], fa_ref[...], (((1,), (0,)), ((), ())),
        preferred_element_type=jnp.float32)
    pt = lax.dot_general(
        w_ref[...], ft_ref[...], (((1,), (0,)), ((), ())),
        preferred_element_type=jnp.float32)
    p = jnp.concatenate([pm, pt], axis=1)
    v = p[0:1, 0:B] + p[1:2, 1:B + 1] + p[2:3, 2:B + 2]
    out_ref[...] = v + b_ref[...]


def _conv_vals(flowT, wk, b2, E):
    nb = E // B
    out = pl.pallas_call(
        _conv_body,
        grid=(nb,),
        in_specs=[
            pl.BlockSpec((12, B), lambda i: (0, i)),
            pl.BlockSpec((12, 128), lambda i: (0, (i + 1) * (B // 128))),
            pl.BlockSpec((3, 12), lambda i: (0, 0)),
            pl.BlockSpec((1, 1), lambda i: (0, 0)),
        ],
        out_specs=pl.BlockSpec((1, B), lambda i: (0, i)),
        out_shape=jax.ShapeDtypeStruct((1, E), jnp.float32),
    )(flowT, flowT, wk, b2)
    return out.reshape(E)


def _sc_scatter(conc, size, src, dst, vals, N2):
    N = conc.shape[0]
    E = src.shape[0]
    ep = E // NW

    mesh = plsc.VectorSubcoreMesh(
        core_axis_name="c", subcore_axis_name="s",
        num_cores=NC, num_subcores=NS)

    UNROLL = 5
    n_chunks = ep // LANES
    assert n_chunks % UNROLL == 0
    nz = N2 // LANES
    assert nz % UNROLL == 0

    def body(conc_hbm, size_hbm, src_hbm, dst_hbm, vals_hbm, part_hbm,
             conc_v, size_v, acc_v, src_v, dst_v, vals_v, sems):
        wid = lax.axis_index("s") * NC + lax.axis_index("c")
        base = wid * ep
        cps = [
            pltpu.async_copy(conc_hbm, conc_v, sems.at[0]),
            pltpu.async_copy(size_hbm, size_v, sems.at[1]),
            pltpu.async_copy(src_hbm.at[pl.ds(base, ep)], src_v, sems.at[2]),
            pltpu.async_copy(dst_hbm.at[pl.ds(base, ep)], dst_v, sems.at[3]),
            pltpu.async_copy(vals_hbm.at[pl.ds(base, ep)], vals_v, sems.at[4]),
        ]

        @plsc.parallel_loop(0, nz, step=1, unroll=UNROLL)
        def zero_body(i):
            acc_v[pl.ds(i * LANES, LANES)] = jnp.zeros((LANES,), jnp.float32)

        for cp in cps:
            cp.wait()

        @plsc.parallel_loop(0, n_chunks, step=1, unroll=UNROLL)
        def edge_body(i):
            sl = pl.ds(i * LANES, LANES)
            s = src_v[sl]
            d = dst_v[sl]
            v = vals_v[sl]
            cs = plsc.load_gather(conc_v, [s])
            szs = plsc.load_gather(size_v, [s])
            szd = plsc.load_gather(size_v, [d])
            contrib = jnp.where(s != d, v * cs * TIME_STEP,
                                jnp.zeros((LANES,), jnp.float32))
            plsc.addupdate_scatter(acc_v, [s], -contrib / szs)
            plsc.addupdate_scatter(acc_v, [d], contrib / szd)

        pltpu.sync_copy(acc_v, part_hbm.at[wid])

    fn = pl.kernel(
        body,
        out_type=jax.ShapeDtypeStruct((NW, N2), jnp.float32),
        mesh=mesh,
        compiler_params=pltpu.CompilerParams(needs_layout_passes=False),
        scratch_types=[
            pltpu.VMEM((N,), jnp.float32),
            pltpu.VMEM((N,), jnp.float32),
            pltpu.VMEM((N2,), jnp.float32),
            pltpu.VMEM((ep,), jnp.int32),
            pltpu.VMEM((ep,), jnp.int32),
            pltpu.VMEM((ep,), jnp.float32),
            pltpu.SemaphoreType.DMA((5,)),
        ],
    )
    return fn(conc, size, src, dst, vals)


def _final(partials, conc_p, people_p, size_p, N, N2):
    def body(part_ref, conc_ref, people_ref, size_ref, out_ref):
        nn = jnp.sum(part_ref[...], axis=0, keepdims=True)
        pex = HUMAN_EXHALATION_FLOW * people_ref[...] / size_ref[...]
        idx = lax.broadcasted_iota(jnp.int32, (1, N2), 1)
        mask = jnp.where(idx == N - 1, 0.0, 1.0)
        out_ref[...] = conc_ref[...] + (nn + pex * TIME_STEP) * mask

    return pl.pallas_call(
        body,
        out_shape=jax.ShapeDtypeStruct((1, N2), jnp.float32),
    )(partials, conc_p, people_p, size_p)


def kernel(origin_data, flow, edge_index, conv_w, conv_b):
    N = origin_data.shape[0]
    E = edge_index.shape[1]
    N2 = ((N + 2559) // 2560) * 2560

    conc = origin_data[:, -1, 0]
    people = origin_data[:, -1, 1]
    size = origin_data[:, -1, 2]

    flowT = jnp.transpose(flow.reshape(E + 2, 12)).astype(jnp.bfloat16)
    wk = jnp.transpose(conv_w[0], (1, 2, 0)).reshape(3, 12).astype(jnp.bfloat16)
    b2 = conv_b.reshape(1, 1)
    vals = _conv_vals(flowT, wk, b2, E)

    src = edge_index[0]
    dst = edge_index[1]
    partials = _sc_scatter(conc, size, src, dst, vals, N2)

    pad = N2 - N
    conc_p = jnp.pad(conc, (0, pad)).reshape(1, N2)
    people_p = jnp.pad(people, (0, pad)).reshape(1, N2)
    size_p = jnp.pad(size, (0, pad), constant_values=1.0).reshape(1, N2)
    res_p = _final(partials, conc_p, people_p, size_p, N, N2)

    result = res_p[0, :N][:, None]
    edge_feat = jnp.concatenate(
        [edge_index.T.astype(jnp.float32), vals[:, None]], axis=1)
    return (result, edge_feat)


# speedup vs baseline: 27.8584x; 1.0089x over previous
import functools

import jax
import jax.numpy as jnp
from jax import lax
from jax.experimental import pallas as pl
from jax.experimental.pallas import tpu as pltpu
from jax.experimental.pallas import tpu_sc as plsc

HUMAN_EXHALATION_FLOW = 0.0052
TIME_STEP = 1.0

NC = 2
NS = 16
NW = NC * NS
LANES = 16

B = 128000


def _conv_body(fa_ref, ft_ref, w_ref, b_ref, out_ref):
    pm = lax.dot_general(
        w_ref[...], fa_ref[...], (((1,), (0,)), ((), ())),
        preferred_element_type=jnp.float32)
    pt = lax.dot_general(
        w_ref[...], ft_ref[...], (((1,), (0,)), ((), ())),
        preferred_element_type=jnp.float32)
    p = jnp.concatenate([pm, pt], axis=1)
    v = p[0:1, 0:B] + p[1:2, 1:B + 1] + p[2:3, 2:B + 2]
    out_ref[...] = v + b_ref[...]


def _conv_vals(flowT, wk, b2, E):
    nb = E // B
    out = pl.pallas_call(
        _conv_body,
        grid=(nb,),
        in_specs=[
            pl.BlockSpec((12, B), lambda i: (0, i)),
            pl.BlockSpec((12, 128), lambda i: (0, (i + 1) * (B // 128))),
            pl.BlockSpec((3, 12), lambda i: (0, 0)),
            pl.BlockSpec((1, 1), lambda i: (0, 0)),
        ],
        out_specs=pl.BlockSpec((1, B), lambda i: (0, i)),
        out_shape=jax.ShapeDtypeStruct((1, E), jnp.float32),
    )(flowT, flowT, wk, b2)
    return out.reshape(E)


def _sc_scatter(conc, size, src, dst, vals, N2):
    N = conc.shape[0]
    E = src.shape[0]
    ep = E // NW

    mesh = plsc.VectorSubcoreMesh(
        core_axis_name="c", subcore_axis_name="s",
        num_cores=NC, num_subcores=NS)

    UNROLL = 5
    n_chunks = ep // LANES
    assert n_chunks % UNROLL == 0
    nz = N2 // LANES
    assert nz % UNROLL == 0

    def body(conc_hbm, size_hbm, src_hbm, dst_hbm, vals_hbm, part_hbm,
             conc_v, size_v, acc_v, src_v, dst_v, vals_v, sems):
        wid = lax.axis_index("s") * NC + lax.axis_index("c")
        base = wid * ep
        cps = [
            pltpu.async_copy(conc_hbm, conc_v, sems.at[0]),
            pltpu.async_copy(size_hbm, size_v, sems.at[1]),
            pltpu.async_copy(src_hbm.at[pl.ds(base, ep)], src_v, sems.at[2]),
            pltpu.async_copy(dst_hbm.at[pl.ds(base, ep)], dst_v, sems.at[3]),
            pltpu.async_copy(vals_hbm.at[pl.ds(base, ep)], vals_v, sems.at[4]),
        ]

        @plsc.parallel_loop(0, nz, step=1, unroll=UNROLL)
        def zero_body(i):
            acc_v[pl.ds(i * LANES, LANES)] = jnp.zeros((LANES,), jnp.float32)

        for cp in cps:
            cp.wait()

        @plsc.parallel_loop(0, n_chunks, step=1, unroll=UNROLL)
        def edge_body(i):
            sl = pl.ds(i * LANES, LANES)
            s = src_v[sl]
            d = dst_v[sl]
            v = vals_v[sl]
            cs = plsc.load_gather(conc_v, [s])
            szs = plsc.load_gather(size_v, [s])
            szd = plsc.load_gather(size_v, [d])
            contrib = jnp.where(s != d, v * cs * TIME_STEP,
                                jnp.zeros((LANES,), jnp.float32))
            plsc.addupdate_scatter(acc_v, [s], -contrib / szs)
            plsc.addupdate_scatter(acc_v, [d], contrib / szd)

        pltpu.sync_copy(acc_v, part_hbm.at[wid])

    fn = pl.kernel(
        body,
        out_type=jax.ShapeDtypeStruct((NW, N2), jnp.float32),
        mesh=mesh,
        compiler_params=pltpu.CompilerParams(needs_layout_passes=False),
        scratch_types=[
            pltpu.VMEM((N,), jnp.float32),
            pltpu.VMEM((N,), jnp.float32),
            pltpu.VMEM((N2,), jnp.float32),
            pltpu.VMEM((ep,), jnp.int32),
            pltpu.VMEM((ep,), jnp.int32),
            pltpu.VMEM((ep,), jnp.float32),
            pltpu.SemaphoreType.DMA((5,)),
        ],
    )
    return fn(conc, size, src, dst, vals)


def _final(partials, conc_p, people_p, size_p, N, N2):
    def body(part_ref, conc_ref, people_ref, size_ref, out_ref):
        nn = jnp.sum(part_ref[...], axis=0, keepdims=True)
        pex = HUMAN_EXHALATION_FLOW * people_ref[...] / size_ref[...]
        idx = lax.broadcasted_iota(jnp.int32, (1, N2), 1)
        mask = jnp.where(idx == N - 1, 0.0, 1.0)
        out_ref[...] = conc_ref[...] + (nn + pex * TIME_STEP) * mask

    return pl.pallas_call(
        body,
        out_shape=jax.ShapeDtypeStruct((1, N2), jnp.float32),
    )(partials, conc_p, people_p, size_p)


def kernel(origin_data, flow, edge_index, conv_w, conv_b):
    N = origin_data.shape[0]
    E = edge_index.shape[1]
    N2 = ((N + 2559) // 2560) * 2560

    conc = origin_data[:, -1, 0]
    people = origin_data[:, -1, 1]
    size = origin_data[:, -1, 2]

    flowT = jnp.transpose(flow.reshape(E + 2, 12)).astype(jnp.bfloat16)
    wk = jnp.transpose(conv_w[0], (1, 2, 0)).reshape(3, 12).astype(jnp.bfloat16)
    b2 = conv_b.reshape(1, 1)
    vals = _conv_vals(flowT, wk, b2, E)

    src = edge_index[0]
    dst = edge_index[1]
    partials = _sc_scatter(conc, size, src, dst, vals, N2)

    pad = N2 - N
    conc_p = jnp.pad(conc, (0, pad)).reshape(1, N2)
    people_p = jnp.pad(people, (0, pad)).reshape(1, N2)
    size_p = jnp.pad(size, (0, pad), constant_values=1.0).reshape(1, N2)
    res_p = _final(partials, conc_p, people_p, size_p, N, N2)

    result = res_p[0, :N][:, None]
    edge_feat = jnp.concatenate(
        [edge_index.T.astype(jnp.float32), vals[:, None]], axis=1)
    return (result, edge_feat)
